# baseline (speedup 1.0000x reference)
import functools

import jax
import jax.numpy as jnp
from jax import lax
from jax.experimental import pallas as pl
from jax.experimental.pallas import tpu as pltpu
from jax.experimental.pallas import tpu_sc as plsc

B = 16384
D = 64
N = 100000
NC = 2
NS = 16
NW = NC * NS
BPW = B // NW
L = 16
SCALE = 0.25

TBLK = 512
TGRID = (N + TBLK - 1) // TBLK


@functools.partial(
    pl.pallas_call,
    grid=(TGRID,),
    in_specs=[
        pl.BlockSpec((TBLK, D), lambda i: (i, 0)),
        pl.BlockSpec((TBLK, D), lambda i: (i, 0)),
    ],
    out_specs=[
        pl.BlockSpec((TBLK // 2, 2 * D), lambda i: (i, 0)),
        pl.BlockSpec((TBLK // 2, 2 * D), lambda i: (i, 0)),
    ],
    out_shape=(
        jax.ShapeDtypeStruct((N // 2, 2 * D), jnp.float32),
        jax.ShapeDtypeStruct((N // 2, 2 * D), jnp.float32),
    ),
)
def _untile(ut_ref, it_ref, ou_ref, oi_ref):
    def merge(x):
        y = x.reshape(TBLK // 2, 2, D)
        return jnp.concatenate([y[:, 0, :], y[:, 1, :]], axis=1)

    ou_ref[...] = merge(ut_ref[...])
    oi_ref[...] = merge(it_ref[...])


@functools.partial(
    pl.kernel,
    out_type=(
        jax.ShapeDtypeStruct((B, D), jnp.float32),
        jax.ShapeDtypeStruct((B, D), jnp.float32),
    ),
    mesh=plsc.VectorSubcoreMesh(core_axis_name="c", subcore_axis_name="s"),
    scratch_types=[
        pltpu.VMEM((BPW,), jnp.int32),
        pltpu.VMEM((BPW, D), jnp.float32),
        pltpu.SemaphoreType.DMA,
    ],
    compiler_params=pltpu.CompilerParams(
        use_tc_tiling_on_sc=False, needs_layout_passes=False),
)
def _gather_scale(users_hbm, items_hbm, utab_hbm, itab_hbm,
                  out_u_hbm, out_i_hbm, idx_v, rows_v, sem):
    wid = lax.axis_index("s") * NC + lax.axis_index("c")
    base = wid * BPW

    def one_table(src_idx_hbm, tab_hbm, out_hbm):
        pltpu.sync_copy(src_idx_hbm.at[pl.ds(base, BPW)], idx_v)
        pltpu.async_copy(tab_hbm.at[idx_v], rows_v, sem).wait()

        def scale_row(i, _):
            for j in range(D // L):
                sl = pl.ds(j * L, L)
                rows_v[i, sl] = rows_v[i, sl] * SCALE
            return 0

        lax.fori_loop(0, BPW, scale_row, 0)
        pltpu.sync_copy(rows_v, out_hbm.at[pl.ds(base, BPW)])

    one_table(users_hbm, utab_hbm, out_u_hbm)
    one_table(items_hbm, itab_hbm, out_i_hbm)


def kernel(users, items, user_table, item_table, edge_user, edge_item):
    del edge_user, edge_item
    u_lin, i_lin = _untile(user_table, item_table)
    utab = u_lin.reshape(N, D)
    itab = i_lin.reshape(N, D)
    return _gather_scale(users, items, utab, itab)


# speedup vs baseline: 1.6889x; 1.6889x over previous
import functools

import jax
import jax.numpy as jnp
from jax import lax
from jax.experimental import pallas as pl
from jax.experimental.pallas import tpu as pltpu
from jax.experimental.pallas import tpu_sc as plsc

B = 16384
D = 64
NC = 2
NS = 16
NW = NC * NS
BPW = B // NW
L = 16
SCALE = 0.25


@functools.partial(
    pl.kernel,
    out_type=(
        jax.ShapeDtypeStruct((B, D), jnp.float32),
        jax.ShapeDtypeStruct((B, D), jnp.float32),
    ),
    mesh=plsc.VectorSubcoreMesh(core_axis_name="c", subcore_axis_name="s"),
    scratch_types=[
        pltpu.VMEM((BPW,), jnp.int32),
        pltpu.VMEM((BPW,), jnp.int32),
        pltpu.VMEM((BPW, D), jnp.float32),
        pltpu.VMEM((BPW, D), jnp.float32),
        pltpu.SemaphoreType.DMA,
        pltpu.SemaphoreType.DMA,
        pltpu.SemaphoreType.DMA,
    ],
    compiler_params=pltpu.CompilerParams(use_tc_tiling_on_sc=False),
)
def _gather_scale(users_hbm, items_hbm, utab_hbm, itab_hbm,
                  out_u_hbm, out_i_hbm,
                  idx_u, idx_i, rows_u, rows_i, sem_u, sem_i, sem_w):
    wid = lax.axis_index("s") * NC + lax.axis_index("c")
    base = wid * BPW

    pltpu.sync_copy(users_hbm.at[pl.ds(base, BPW)], idx_u)
    pltpu.sync_copy(items_hbm.at[pl.ds(base, BPW)], idx_i)
    cp_u = pltpu.async_copy(utab_hbm.at[idx_u], rows_u, sem_u)
    cp_i = pltpu.async_copy(itab_hbm.at[idx_i], rows_i, sem_i)

    def scale_in(rows_v):
        def scale_row(i, _):
            for j in range(D // L):
                sl = pl.ds(j * L, L)
                rows_v[i, sl] = rows_v[i, sl] * SCALE
            return 0

        lax.fori_loop(0, BPW, scale_row, 0)

    cp_u.wait()
    scale_in(rows_u)
    w_u = pltpu.async_copy(rows_u, out_u_hbm.at[pl.ds(base, BPW)], sem_w)
    cp_i.wait()
    scale_in(rows_i)
    w_i = pltpu.async_copy(rows_i, out_i_hbm.at[pl.ds(base, BPW)], sem_w)
    w_u.wait()
    w_i.wait()


def kernel(users, items, user_table, item_table, edge_user, edge_item):
    del edge_user, edge_item
    return _gather_scale(users, items, user_table, item_table)


# speedup vs baseline: 1.7056x; 1.0099x over previous
import functools

import jax
import jax.numpy as jnp
from jax import lax
from jax.experimental import pallas as pl
from jax.experimental.pallas import tpu as pltpu
from jax.experimental.pallas import tpu_sc as plsc

B = 16384
D = 64
NC = 2
NS = 16
NW = NC * NS
BPW = B // NW
L = 16
SCALE = 0.25


@functools.partial(
    pl.kernel,
    out_type=(
        jax.ShapeDtypeStruct((B, D), jnp.float32),
        jax.ShapeDtypeStruct((B, D), jnp.float32),
    ),
    mesh=plsc.VectorSubcoreMesh(core_axis_name="c", subcore_axis_name="s"),
    scratch_types=[
        pltpu.VMEM((BPW,), jnp.int32),
        pltpu.VMEM((BPW,), jnp.int32),
        pltpu.VMEM((BPW, D), jnp.float32),
        pltpu.VMEM((BPW, D), jnp.float32),
        pltpu.SemaphoreType.DMA,
        pltpu.SemaphoreType.DMA,
        pltpu.SemaphoreType.DMA,
    ],
    compiler_params=pltpu.CompilerParams(use_tc_tiling_on_sc=False),
)
def _gather_scale(users_hbm, items_hbm, utab_hbm, itab_hbm,
                  out_u_hbm, out_i_hbm,
                  idx_u, idx_i, rows_u, rows_i, sem_u, sem_i, sem_w):
    wid = lax.axis_index("s") * NC + lax.axis_index("c")
    base = wid * BPW

    pltpu.sync_copy(users_hbm.at[pl.ds(base, BPW)], idx_u)
    pltpu.sync_copy(items_hbm.at[pl.ds(base, BPW)], idx_i)
    cp_u = pltpu.async_copy(utab_hbm.at[idx_u], rows_u, sem_u)
    cp_i = pltpu.async_copy(itab_hbm.at[idx_i], rows_i, sem_i)

    def scale_in(rows_v):
        def scale_row(i, _):
            for j in range(D // L):
                sl = pl.ds(j * L, L)
                rows_v[i, sl] = rows_v[i, sl] * SCALE
            return 0

        lax.fori_loop(0, BPW, scale_row, 0, unroll=4)

    cp_u.wait()
    scale_in(rows_u)
    w_u = pltpu.async_copy(rows_u, out_u_hbm.at[pl.ds(base, BPW)], sem_w)
    cp_i.wait()
    scale_in(rows_i)
    w_i = pltpu.async_copy(rows_i, out_i_hbm.at[pl.ds(base, BPW)], sem_w)
    w_u.wait()
    w_i.wait()


def kernel(users, items, user_table, item_table, edge_user, edge_item):
    del edge_user, edge_item
    return _gather_scale(users, items, user_table, item_table)


# speedup vs baseline: 1.7090x; 1.0020x over previous
import functools

import jax
import jax.numpy as jnp
from jax import lax
from jax.experimental import pallas as pl
from jax.experimental.pallas import tpu as pltpu
from jax.experimental.pallas import tpu_sc as plsc

B = 16384
D = 64
NC = 2
NS = 16
NW = NC * NS
BPW = B // NW
L = 16
SCALE = 0.25


@functools.partial(
    pl.kernel,
    out_type=(
        jax.ShapeDtypeStruct((B, D), jnp.float32),
        jax.ShapeDtypeStruct((B, D), jnp.float32),
    ),
    mesh=plsc.VectorSubcoreMesh(core_axis_name="c", subcore_axis_name="s"),
    scratch_types=[
        pltpu.VMEM((BPW,), jnp.int32),
        pltpu.VMEM((BPW,), jnp.int32),
        pltpu.VMEM((BPW, D), jnp.float32),
        pltpu.VMEM((BPW, D), jnp.float32),
        pltpu.SemaphoreType.DMA,
        pltpu.SemaphoreType.DMA,
        pltpu.SemaphoreType.DMA,
    ],
    compiler_params=pltpu.CompilerParams(use_tc_tiling_on_sc=False),
)
def _gather_scale(users_hbm, items_hbm, utab_hbm, itab_hbm,
                  out_u_hbm, out_i_hbm,
                  idx_u, idx_i, rows_u, rows_i, sem_u, sem_i, sem_w):
    wid = lax.axis_index("s") * NC + lax.axis_index("c")
    base = wid * BPW

    pltpu.sync_copy(users_hbm.at[pl.ds(base, BPW)], idx_u)
    pltpu.sync_copy(items_hbm.at[pl.ds(base, BPW)], idx_i)
    cp_u = pltpu.async_copy(utab_hbm.at[idx_u], rows_u, sem_u)
    cp_i = pltpu.async_copy(itab_hbm.at[idx_i], rows_i, sem_i)

    def scale_in(rows_v):
        def scale_row(i, _):
            for j in range(D // L):
                sl = pl.ds(j * L, L)
                rows_v[i, sl] = rows_v[i, sl] * SCALE
            return 0

        lax.fori_loop(0, BPW, scale_row, 0, unroll=8)

    cp_u.wait()
    scale_in(rows_u)
    w_u = pltpu.async_copy(rows_u, out_u_hbm.at[pl.ds(base, BPW)], sem_w)
    cp_i.wait()
    scale_in(rows_i)
    w_i = pltpu.async_copy(rows_i, out_i_hbm.at[pl.ds(base, BPW)], sem_w)
    w_u.wait()
    w_i.wait()


def kernel(users, items, user_table, item_table, edge_user, edge_item):
    del edge_user, edge_item
    return _gather_scale(users, items, user_table, item_table)
